# Initial kernel scaffold; baseline (speedup 1.0000x reference)
import jax
import jax.numpy as jnp
from jax.experimental import pallas as pl


def kernel(daytime, W_day, W_time):
    raise NotImplementedError("write your pallas kernel here")


# speedup vs baseline: 4.2753x; 4.2753x over previous
import functools

import jax
import jax.numpy as jnp
from jax import lax
from jax.experimental import pallas as pl
from jax.experimental.pallas import tpu as pltpu
from jax.experimental.pallas import tpu_sc as plsc

TIME_ROWS = 1440
ROW = 64
NC, NS, L = 2, 16, 16
NW = NC * NS

N = 4096 * 50
FN = 2 * N
PW = FN // NW
CHUNK = 1280
NCHUNK = PW // CHUNK
NSTREAM = CHUNK // 128


def _sc_gather(dt_flat, wcat):
    mesh = plsc.VectorSubcoreMesh(core_axis_name="c", subcore_axis_name="s")

    @functools.partial(
        pl.kernel,
        out_type=jax.ShapeDtypeStruct((FN, ROW), jnp.float32),
        mesh=mesh,
        scratch_types=[
            pltpu.VMEM((CHUNK,), jnp.int32),
            pltpu.VMEM((NSTREAM, 128), jnp.int32),
            pltpu.VMEM((CHUNK, ROW), jnp.float32),
            pltpu.SemaphoreType.DMA,
        ],
        compiler_params=pltpu.CompilerParams(use_tc_tiling_on_sc=False),
    )
    def k(dt_hbm, wcat_hbm, out_hbm, rawbuf, fusedbuf, obuf, sem):
        wid = lax.axis_index("s") * NC + lax.axis_index("c")
        lane = lax.iota(jnp.int32, L)
        swap = lane ^ 1
        addv = (lane & 1) * TIME_ROWS

        def chunk_body(c, _):
            fbase = wid * PW + c * CHUNK
            pltpu.sync_copy(dt_hbm.at[pl.ds(fbase, CHUNK)], rawbuf)

            def fuse_body(t, _):
                v = rawbuf[pl.ds(t * L, L)]
                v = v.at[swap].get(mode="promise_in_bounds") + addv
                fusedbuf[t // 8, pl.ds((t % 8) * L, L)] = v
                return _

            lax.fori_loop(0, CHUNK // L, fuse_body, None)

            copies = [
                pltpu.async_copy(
                    wcat_hbm.at[fusedbuf.at[j]],
                    obuf.at[pl.ds(j * 128, 128)],
                    sem,
                )
                for j in range(NSTREAM)
            ]
            for cp in copies:
                cp.wait()

            pltpu.sync_copy(obuf, out_hbm.at[pl.ds(fbase, CHUNK)])
            return _

        lax.fori_loop(0, NCHUNK, chunk_body, None)

    return k(dt_flat, wcat)


def kernel(daytime, W_day, W_time):
    b, s, _ = daytime.shape
    dt_flat = daytime.astype(jnp.int32).reshape(2 * b * s)
    wcat = jnp.concatenate([W_time, W_day], axis=0)
    out2 = _sc_gather(dt_flat, wcat)
    return out2.reshape(b, s, 2 * ROW)


# speedup vs baseline: 4.2810x; 1.0013x over previous
import functools

import jax
import jax.numpy as jnp
from jax import lax
from jax.experimental import pallas as pl
from jax.experimental.pallas import tpu as pltpu
from jax.experimental.pallas import tpu_sc as plsc

TIME_ROWS = 1440
ROW = 64
NC, NS, L = 2, 16, 16
NW = NC * NS

N = 4096 * 50
FN = 2 * N
PW = FN // NW
SLOT = 128
NSLOT = PW // SLOT
SPC = 5
CH = SLOT * SPC
NCH = PW // CH


def _sc_gather(dt_flat, wcat):
    mesh = plsc.VectorSubcoreMesh(core_axis_name="c", subcore_axis_name="s")

    @functools.partial(
        pl.kernel,
        out_type=jax.ShapeDtypeStruct((FN, ROW), jnp.float32),
        mesh=mesh,
        scratch_types=[
            pltpu.VMEM((PW,), jnp.int32),
            pltpu.VMEM((NSLOT, SLOT), jnp.int32),
            pltpu.VMEM((CH, ROW), jnp.float32),
            pltpu.VMEM((CH, ROW), jnp.float32),
            pltpu.SemaphoreType.DMA,
            pltpu.SemaphoreType.DMA,
        ],
        compiler_params=pltpu.CompilerParams(use_tc_tiling_on_sc=False),
    )
    def k(dt_hbm, wcat_hbm, out_hbm, rawbuf, fusedbuf, bufa, bufb, gsem, wsem):
        wid = lax.axis_index("s") * NC + lax.axis_index("c")
        base = wid * PW
        lane = lax.iota(jnp.int32, L)
        swap = lane ^ 1
        addv = (lane & 1) * TIME_ROWS

        pltpu.sync_copy(dt_hbm.at[pl.ds(base, PW)], rawbuf)

        def fuse_body(t, _):
            for u in range(SLOT // L):
                v = rawbuf[pl.ds((t * (SLOT // L) + u) * L, L)]
                v = v.at[swap].get(mode="promise_in_bounds") + addv
                fusedbuf[t, pl.ds(u * L, L)] = v
            return _

        lax.fori_loop(0, NSLOT, fuse_body, None)

        def fire_gathers(c, buf):
            for j in range(SPC):
                pltpu.async_copy(
                    wcat_hbm.at[fusedbuf.at[c * SPC + j]],
                    buf.at[pl.ds(j * SLOT, SLOT)],
                    gsem,
                )

        def drain_gathers(buf):
            for j in range(SPC):
                pltpu.make_async_copy(
                    wcat_hbm.at[fusedbuf.at[0]],
                    buf.at[pl.ds(j * SLOT, SLOT)],
                    gsem,
                ).wait()

        def out_slice(c):
            return out_hbm.at[pl.ds(base + c * CH, CH)]

        fire_gathers(0, bufa)

        def outer(c2, _):
            for b in range(2):
                buf, other = (bufa, bufb) if b == 0 else (bufb, bufa)
                c = c2 * 2 + b
                drain_gathers(buf)

                @pl.when(c > 0)
                def _():
                    pltpu.make_async_copy(other, out_slice(c - 1), wsem).wait()

                pltpu.async_copy(buf, out_slice(c), wsem)

                @pl.when(c < NCH - 1)
                def _():
                    fire_gathers(c + 1, other)

            return _

        lax.fori_loop(0, NCH // 2, outer, None)
        pltpu.make_async_copy(bufb, out_slice(NCH - 1), wsem).wait()

    return k(dt_flat, wcat)


def kernel(daytime, W_day, W_time):
    b, s, _ = daytime.shape
    dt_flat = daytime.astype(jnp.int32).reshape(2 * b * s)
    wcat = jnp.concatenate([W_time, W_day], axis=0)
    out2 = _sc_gather(dt_flat, wcat)
    return out2.reshape(b, s, 2 * ROW)
